# baseline (speedup 1.0000x reference)
import functools

import jax
import jax.numpy as jnp
from jax import lax
from jax.experimental import pallas as pl
from jax.experimental.pallas import tpu as pltpu
from jax.experimental.pallas import tpu_sc as plsc

_NC = 2
_NS = 16
_NW = _NC * _NS
_IDXW = 100


@functools.partial(jax.jit, static_argnames=("seq", "dim"))
def _embed(ids3d, token_table, position_table, seq, dim):
    batch = ids3d.shape[0]
    nsplit = ids3d.shape[1]
    rows_per_w = batch // _NW
    mesh = plsc.VectorSubcoreMesh(core_axis_name="c", subcore_axis_name="s")

    @functools.partial(
        pl.kernel,
        out_type=jax.ShapeDtypeStruct((batch, seq, dim), jnp.float32),
        mesh=mesh,
        compiler_params=pltpu.CompilerParams(use_tc_tiling_on_sc=False),
        scratch_types=[
            pltpu.VMEM((nsplit, _IDXW), jnp.int32),
            pltpu.VMEM((seq, dim), jnp.float32),
            pltpu.VMEM((seq, dim), jnp.float32),
            pltpu.SemaphoreType.DMA,
        ],
    )
    def k(ids_hbm, tok_hbm, pos_hbm, out_hbm, idx_v, rows_v, pos_v, sem):
        wid = lax.axis_index("s") * _NC + lax.axis_index("c")
        b0 = wid * rows_per_w

        pltpu.sync_copy(pos_hbm.at[pl.ds(0, seq)], pos_v)

        def step(t, carry):
            b = b0 + t
            pltpu.sync_copy(ids_hbm.at[b], idx_v)
            cps = [
                pltpu.async_copy(tok_hbm.at[idx_v.at[h]],
                                 rows_v.at[pl.ds(h * _IDXW, _IDXW)], sem)
                for h in range(nsplit)
            ]
            for cp in cps:
                cp.wait()

            def addrow(i, c):
                for j in range(dim // 16):
                    sl = pl.ds(j * 16, 16)
                    rows_v[i, sl] = rows_v[i, sl] + pos_v[i, sl]
                return c

            lax.fori_loop(0, seq, addrow, 0)
            pltpu.sync_copy(rows_v, out_hbm.at[b])
            return carry

        lax.fori_loop(0, rows_per_w, step, 0)

    return k(ids3d, token_table, position_table)


def kernel(input_ids, token_table, position_table):
    b, s = input_ids.shape
    dim = token_table.shape[1]
    assert b % _NW == 0 and s % _IDXW == 0
    ids3d = input_ids.astype(jnp.int32).reshape(b, s // _IDXW, _IDXW)
    out = _embed(ids3d, token_table, position_table, s, dim)
    return out


# speedup vs baseline: 1.1651x; 1.1651x over previous
import functools

import jax
import jax.numpy as jnp
from jax import lax
from jax.experimental import pallas as pl
from jax.experimental.pallas import tpu as pltpu
from jax.experimental.pallas import tpu_sc as plsc

_NC = 2
_NS = 16
_NW = _NC * _NS
_IDXW = 100


@functools.partial(jax.jit, static_argnames=("seq", "dim"))
def _embed(ids3d, token_table, position_table, seq, dim):
    batch = ids3d.shape[0]
    nsplit = ids3d.shape[1]
    rows_per_w = batch // _NW
    mesh = plsc.VectorSubcoreMesh(core_axis_name="c", subcore_axis_name="s")

    @functools.partial(
        pl.kernel,
        out_type=jax.ShapeDtypeStruct((batch, seq, dim), jnp.float32),
        mesh=mesh,
        compiler_params=pltpu.CompilerParams(use_tc_tiling_on_sc=False),
        scratch_types=[
            pltpu.VMEM((2, nsplit, _IDXW), jnp.int32),
            pltpu.VMEM((2, seq, dim), jnp.float32),
            pltpu.VMEM((seq, dim), jnp.float32),
            pltpu.SemaphoreType.DMA((2,)),
            pltpu.SemaphoreType.DMA((2,)),
        ],
    )
    def k(ids_hbm, tok_hbm, pos_hbm, out_hbm, idx_v, rows_v, pos_v,
          gsem, isem):
        wid = lax.axis_index("s") * _NC + lax.axis_index("c")
        b0 = wid * rows_per_w

        pltpu.sync_copy(pos_hbm.at[pl.ds(0, seq)], pos_v)

        def launch_gather(slot):
            for h in range(nsplit):
                pltpu.async_copy(
                    tok_hbm.at[idx_v.at[slot, h]],
                    rows_v.at[slot, pl.ds(h * _IDXW, _IDXW)],
                    gsem.at[slot])

        def wait_gather(slot):
            pltpu.make_async_copy(
                tok_hbm.at[pl.ds(0, seq)], rows_v.at[slot],
                gsem.at[slot]).wait()

        for b in range(2):
            pltpu.sync_copy(ids_hbm.at[b0 + b], idx_v.at[b])
            launch_gather(b)

        def pair(g, carry):
            t0 = g * 2
            for b in range(2):
                t = t0 + b
                wait_gather(b)

                @pl.when(t + 2 < rows_per_w)
                def _():
                    pltpu.async_copy(ids_hbm.at[b0 + t + 2], idx_v.at[b],
                                     isem.at[b])

                @plsc.parallel_loop(0, seq, unroll=4)
                def _add(i):
                    for j in range(dim // 16):
                        sl = pl.ds(j * 16, 16)
                        rows_v[b, i, sl] = rows_v[b, i, sl] + pos_v[i, sl]

                pltpu.sync_copy(rows_v.at[b], out_hbm.at[b0 + t])

                @pl.when(t + 2 < rows_per_w)
                def _():
                    pltpu.make_async_copy(ids_hbm.at[b0 + t + 2],
                                          idx_v.at[b], isem.at[b]).wait()
                    launch_gather(b)

            return carry

        lax.fori_loop(0, rows_per_w // 2, pair, 0)

    return k(ids3d, token_table, position_table)


def kernel(input_ids, token_table, position_table):
    b, s = input_ids.shape
    dim = token_table.shape[1]
    assert b % (2 * _NW) == 0 and s % _IDXW == 0
    ids3d = input_ids.astype(jnp.int32).reshape(b, s // _IDXW, _IDXW)
    out = _embed(ids3d, token_table, position_table, s, dim)
    return out


# speedup vs baseline: 1.2089x; 1.0376x over previous
import functools

import jax
import jax.numpy as jnp
from jax import lax
from jax.experimental import pallas as pl
from jax.experimental.pallas import tpu as pltpu
from jax.experimental.pallas import tpu_sc as plsc

_NC = 2
_NS = 16
_NW = _NC * _NS
_BK = 128


@functools.partial(jax.jit, static_argnames=("seq", "dim"))
def _embed(ids_t, tbl_p, pos_t, seq, dim):
    batch = ids_t.shape[1]
    padw = tbl_p.shape[1]
    maxseq = pos_t.shape[1]
    nsb = seq // 8
    mesh = plsc.VectorSubcoreMesh(core_axis_name="c", subcore_axis_name="s")

    @functools.partial(
        pl.kernel,
        out_type=jax.ShapeDtypeStruct((seq, dim, batch), jnp.float32),
        mesh=mesh,
        compiler_params=pltpu.CompilerParams(use_tc_tiling_on_sc=True,
                                             needs_layout_passes=False),
        scratch_types=[
            pltpu.VMEM((2, 8, _BK), jnp.int32),
            pltpu.VMEM((2, _BK, padw), jnp.float32),
            pltpu.VMEM((2, dim, _BK), jnp.float32),
            pltpu.VMEM((dim, maxseq), jnp.float32),
            pltpu.SemaphoreType.DMA((2,)),
            pltpu.SemaphoreType.DMA((2,)),
            pltpu.SemaphoreType.DMA,
        ],
    )
    def k(ids_hbm, tbl_hbm, pos_hbm, out_hbm, idx_v, g_v, o_v, pos_v,
          gsem, wsem, isem):
        wid = lax.axis_index("s") * _NC + lax.axis_index("c")
        b0 = wid * _BK

        pltpu.sync_copy(pos_hbm, pos_v)
        pltpu.sync_copy(ids_hbm.at[pl.ds(0, 8), pl.ds(b0, _BK)], idx_v.at[0])

        rows = [lax.iota(jnp.int32, 16) + (16 * i) for i in range(_BK // 16)]

        def launch_gather(t, b):
            sb = t // 8
            r = lax.rem(t, 8)
            pltpu.async_copy(
                tbl_hbm.at[idx_v.at[lax.rem(sb, 2), r]], g_v.at[b],
                gsem.at[b])

        def pair(g, carry):
            t0 = g * 2
            for b in range(2):
                t = t0 + b
                sb = t // 8
                r = lax.rem(t, 8)

                pltpu.make_async_copy(
                    tbl_hbm.at[pl.ds(0, _BK)], g_v.at[b], gsem.at[b]).wait()

                @pl.when(jnp.logical_and(r == 0, sb + 1 < nsb))
                def _():
                    pltpu.async_copy(
                        ids_hbm.at[pl.ds((sb + 1) * 8, 8), pl.ds(b0, _BK)],
                        idx_v.at[lax.rem(sb + 1, 2)], isem)

                @pl.when(t >= 2)
                def _():
                    pltpu.make_async_copy(
                        out_hbm.at[0, :, pl.ds(0, _BK)], o_v.at[b],
                        wsem.at[b]).wait()

                tsplat = jnp.full((16,), t, jnp.int32)

                @plsc.parallel_loop(0, dim, unroll=2)
                def _tr(d):
                    col = jnp.full((16,), d, jnp.int32)
                    pv = plsc.load_gather(pos_v, [col, tsplat])
                    for i in range(_BK // 16):
                        vals = plsc.load_gather(g_v.at[b], [rows[i], col])
                        o_v[b, d, pl.ds(i * 16, 16)] = vals + pv

                pltpu.async_copy(
                    o_v.at[b], out_hbm.at[t, :, pl.ds(b0, _BK)], wsem.at[b])

                @pl.when(jnp.logical_and(r == 6, sb + 1 < nsb))
                def _():
                    pltpu.make_async_copy(
                        ids_hbm.at[pl.ds(0, 8), pl.ds(0, _BK)], idx_v.at[0],
                        isem).wait()

                @pl.when(t + 2 < seq)
                def _():
                    launch_gather(t + 2, b)

            return carry

        launch_gather(0, 0)
        launch_gather(1, 1)
        lax.fori_loop(0, seq // 2, pair, 0)

        for b in range(2):
            pltpu.make_async_copy(
                out_hbm.at[0, :, pl.ds(0, _BK)], o_v.at[b], wsem.at[b]).wait()

    return k(ids_t, tbl_p, pos_t)


def kernel(input_ids, token_table, position_table):
    b, s = input_ids.shape
    dim = token_table.shape[1]
    assert b == _NW * _BK and s % 8 == 0 and dim <= 128
    ids_t = input_ids.T.astype(jnp.int32)
    tbl_p = jnp.pad(token_table, ((0, 0), (0, 128 - dim)))
    pos_t = position_table.T
    out_t = _embed(ids_t, tbl_p, pos_t, s, dim)
    return out_t.transpose(2, 0, 1)
